# baseline (speedup 1.0000x reference)
import functools

import jax
import jax.numpy as jnp
from jax.experimental import pallas as pl
from jax.experimental.pallas import tpu as pltpu

_EPS = 1e-5
_VMEM_LIMIT = 64 * 1024 * 1024


def _tap_stack(x, kw, stride):
    L = x.shape[1]
    taps = [x]
    for k in range(1, kw):
        taps.append(pltpu.roll(x, (L - k * stride) % L, 1))
    return jnp.concatenate(taps, axis=0)


def _mask_w(acc, w_img, wo, n):
    lane = jax.lax.broadcasted_iota(jnp.int32, acc.shape, 1)
    return jnp.where((lane // n) % w_img < wo, acc, 0.0)


def _scale_shift(s_ref, q_ref, gb_ref, count):
    c = gb_ref.shape[0]
    total = jnp.sum(s_ref[...], axis=0)
    total_sq = jnp.sum(q_ref[...], axis=0)
    mean = total * (1.0 / count)
    var = jnp.maximum(total_sq * (1.0 / count) - mean * mean, 0.0)
    inv = gb_ref[:, 0:1] * jax.lax.rsqrt(var + _EPS)
    shift = gb_ref[:, 1:2] - mean * inv
    return inv, shift


def _wcat(w_ref):
    o, i, kw = w_ref.shape
    return jnp.transpose(w_ref[...], (0, 2, 1)).reshape(o, kw * i)


def _p1_kernel(x_ref, w_ref, h_ref, s_ref, q_ref, *, kw, w_img, wo, n):
    cin = x_ref.shape[0]
    x = x_ref[...].reshape(cin, -1)
    acc = jnp.dot(_wcat(w_ref), _tap_stack(x, kw, n),
                  preferred_element_type=jnp.float32)
    acc = _mask_w(acc, w_img, wo, n)
    cmid = acc.shape[0]
    h_ref[...] = acc.astype(jnp.bfloat16)
    s_ref[...] = jnp.sum(acc, axis=1, keepdims=True).reshape(1, cmid, 1)
    q_ref[...] = jnp.sum(acc * acc, axis=1, keepdims=True).reshape(1, cmid, 1)


def _p2_kernel(h_ref, s1_ref, q1_ref, gb1_ref, w_ref, s_ref, q_ref,
               *, kw, w_img, wo, n, count1):
    scale, shift = _scale_shift(s1_ref, q1_ref, gb1_ref, count1)
    h = h_ref[...].astype(jnp.float32)
    y = jnp.maximum(h * scale + shift, 0.0)
    acc = jnp.dot(_wcat(w_ref), _tap_stack(y, kw, n),
                  preferred_element_type=jnp.float32)
    acc = _mask_w(acc, w_img, wo, n)
    cout = acc.shape[0]
    s_ref[...] = jnp.sum(acc, axis=1, keepdims=True).reshape(1, cout, 1)
    q_ref[...] = jnp.sum(acc * acc, axis=1, keepdims=True).reshape(1, cout, 1)


def _p3_kernel(h_ref, s1_ref, q1_ref, gb1_ref, w_ref, s2_ref, q2_ref, gb2_ref,
               o_ref, *, kw, w_img, wo, n, hb, count1, count2):
    scale1, shift1 = _scale_shift(s1_ref, q1_ref, gb1_ref, count1)
    scale2, shift2 = _scale_shift(s2_ref, q2_ref, gb2_ref, count2)
    h = h_ref[...].astype(jnp.float32)
    y = jnp.maximum(h * scale1 + shift1, 0.0)
    acc = jnp.dot(_wcat(w_ref), _tap_stack(y, kw, n),
                  preferred_element_type=jnp.float32)
    z = jnp.maximum(acc * scale2 + shift2, 0.0)
    cout = z.shape[0]
    z4 = z.reshape(cout, hb, w_img, n)
    o_ref[...] = jnp.transpose(z4, (0, 2, 1, 3))[:, :wo]


@jax.jit
def _conv_block(x_nchw, w1_oihw, g1, b1, w2_oihw, g2, b2):
    N, Cin, H, W = x_nchw.shape
    Cmid = w1_oihw.shape[0]
    Cout = w2_oihw.shape[0]
    KW = w1_oihw.shape[3]
    Wo1 = W - (KW - 1)
    Wo2 = Wo1 - (KW - 1)
    P = N * H * W
    count1 = float(N * H * Wo1)
    count2 = float(N * H * Wo2)

    HB1 = 8
    nblk1 = H // HB1
    L1 = HB1 * W * N
    HB3 = 8
    nblk3 = H // HB3
    L3 = HB3 * W * N

    x_t = jnp.transpose(x_nchw, (1, 2, 3, 0))
    w1_cat = w1_oihw.reshape(Cmid, Cin, KW)
    w2_cat = w2_oihw.reshape(Cout, Cmid, KW)
    gb1 = jnp.stack([g1, b1], axis=1)
    gb2 = jnp.stack([g2, b2], axis=1)

    cparams = pltpu.CompilerParams(
        dimension_semantics=("parallel",),
        vmem_limit_bytes=_VMEM_LIMIT,
    )

    def stat_out_spec(c):
        return pl.BlockSpec((1, c, 1), lambda i: (i, 0, 0))

    def stat_in_spec(nblk, c):
        return pl.BlockSpec((nblk, c, 1), lambda i: (0, 0, 0))

    def full2d_spec(r, c):
        return pl.BlockSpec((r, c), lambda i: (0, 0))

    h1, s1, q1 = pl.pallas_call(
        functools.partial(_p1_kernel, kw=KW, w_img=W, wo=Wo1, n=N),
        grid=(nblk1,),
        in_specs=[
            pl.BlockSpec((Cin, HB1, W, N), lambda i: (0, i, 0, 0)),
            pl.BlockSpec((Cmid, Cin, KW), lambda i: (0, 0, 0)),
        ],
        out_specs=(
            pl.BlockSpec((Cmid, L1), lambda i: (0, i)),
            stat_out_spec(Cmid),
            stat_out_spec(Cmid),
        ),
        out_shape=(
            jax.ShapeDtypeStruct((Cmid, P), jnp.bfloat16),
            jax.ShapeDtypeStruct((nblk1, Cmid, 1), jnp.float32),
            jax.ShapeDtypeStruct((nblk1, Cmid, 1), jnp.float32),
        ),
        compiler_params=cparams,
    )(x_t, w1_cat)

    s2, q2 = pl.pallas_call(
        functools.partial(_p2_kernel, kw=KW, w_img=W, wo=Wo2, n=N,
                          count1=count1),
        grid=(nblk1,),
        in_specs=[
            pl.BlockSpec((Cmid, L1), lambda i: (0, i)),
            stat_in_spec(nblk1, Cmid),
            stat_in_spec(nblk1, Cmid),
            full2d_spec(Cmid, 2),
            pl.BlockSpec((Cout, Cmid, KW), lambda i: (0, 0, 0)),
        ],
        out_specs=(stat_out_spec(Cout), stat_out_spec(Cout)),
        out_shape=(
            jax.ShapeDtypeStruct((nblk1, Cout, 1), jnp.float32),
            jax.ShapeDtypeStruct((nblk1, Cout, 1), jnp.float32),
        ),
        compiler_params=cparams,
    )(h1, s1, q1, gb1, w2_cat)

    out_t = pl.pallas_call(
        functools.partial(_p3_kernel, kw=KW, w_img=W, wo=Wo2, n=N, hb=HB3,
                          count1=count1, count2=count2),
        grid=(nblk3,),
        in_specs=[
            pl.BlockSpec((Cmid, L3), lambda i: (0, i)),
            stat_in_spec(nblk1, Cmid),
            stat_in_spec(nblk1, Cmid),
            full2d_spec(Cmid, 2),
            pl.BlockSpec((Cout, Cmid, KW), lambda i: (0, 0, 0)),
            stat_in_spec(nblk1, Cout),
            stat_in_spec(nblk1, Cout),
            full2d_spec(Cout, 2),
        ],
        out_specs=pl.BlockSpec((Cout, Wo2, HB3, N), lambda i: (0, 0, i, 0)),
        out_shape=jax.ShapeDtypeStruct((Cout, Wo2, H, N), jnp.float32),
        compiler_params=cparams,
    )(h1, s1, q1, gb1, w2_cat, s2, q2, gb2)

    return jnp.transpose(out_t, (3, 0, 2, 1))


def kernel(x_nchw, w1_oihw, g1, b1, w2_oihw, g2, b2):
    return _conv_block(x_nchw, w1_oihw, g1, b1, w2_oihw, g2, b2)


# speedup vs baseline: 1.0151x; 1.0151x over previous
import functools

import jax
import jax.numpy as jnp
from jax.experimental import pallas as pl
from jax.experimental.pallas import tpu as pltpu

_EPS = 1e-5
_VMEM_LIMIT = 64 * 1024 * 1024


def _tap_stack(x, kw, stride):
    L = x.shape[1]
    taps = [x]
    for k in range(1, kw):
        taps.append(pltpu.roll(x, (L - k * stride) % L, 1))
    return jnp.concatenate(taps, axis=0)


def _mask_w(acc, w_img, wo, n):
    lane = jax.lax.broadcasted_iota(jnp.int32, acc.shape, 1)
    return jnp.where((lane // n) % w_img < wo, acc, 0.0)


def _scale_shift(s_ref, q_ref, gb_ref, count):
    c = gb_ref.shape[0]
    total = jnp.sum(s_ref[...], axis=0)
    total_sq = jnp.sum(q_ref[...], axis=0)
    mean = total * (1.0 / count)
    var = jnp.maximum(total_sq * (1.0 / count) - mean * mean, 0.0)
    inv = gb_ref[:, 0:1] * jax.lax.rsqrt(var + _EPS)
    shift = gb_ref[:, 1:2] - mean * inv
    return inv, shift


def _p1_kernel(x_ref, w_ref, h_ref, s_ref, q_ref, *, kw, w_img, wo, n):
    cin = x_ref.shape[0]
    x = x_ref[...].reshape(cin, -1)
    acc = jnp.dot(w_ref[...], _tap_stack(x, kw, n),
                  preferred_element_type=jnp.float32)
    acc = _mask_w(acc, w_img, wo, n)
    cmid = acc.shape[0]
    h_ref[...] = acc.astype(jnp.bfloat16)
    s_ref[...] = jnp.sum(acc, axis=1, keepdims=True).reshape(1, cmid, 1)
    q_ref[...] = jnp.sum(acc * acc, axis=1, keepdims=True).reshape(1, cmid, 1)


def _p2_kernel(h_ref, s1_ref, q1_ref, gb1_ref, w_ref, s_ref, q_ref,
               *, kw, w_img, wo, n, count1):
    scale, shift = _scale_shift(s1_ref, q1_ref, gb1_ref, count1)
    h = h_ref[...].astype(jnp.float32)
    y = jnp.maximum(h * scale + shift, 0.0)
    acc = jnp.dot(w_ref[...], _tap_stack(y, kw, n),
                  preferred_element_type=jnp.float32)
    acc = _mask_w(acc, w_img, wo, n)
    cout = acc.shape[0]
    s_ref[...] = jnp.sum(acc, axis=1, keepdims=True).reshape(1, cout, 1)
    q_ref[...] = jnp.sum(acc * acc, axis=1, keepdims=True).reshape(1, cout, 1)


def _p3_kernel(h_ref, s1_ref, q1_ref, gb1_ref, w_ref, s2_ref, q2_ref, gb2_ref,
               o_ref, *, kw, w_img, wo, n, hb, count1, count2):
    scale1, shift1 = _scale_shift(s1_ref, q1_ref, gb1_ref, count1)
    scale2, shift2 = _scale_shift(s2_ref, q2_ref, gb2_ref, count2)
    h = h_ref[...].astype(jnp.float32)
    y = jnp.maximum(h * scale1 + shift1, 0.0)
    acc = jnp.dot(w_ref[...], _tap_stack(y, kw, n),
                  preferred_element_type=jnp.float32)
    z = jnp.maximum(acc * scale2 + shift2, 0.0)
    cout = z.shape[0]
    z4 = z.reshape(cout, hb, w_img, n)
    o_ref[...] = jnp.transpose(z4, (0, 2, 1, 3))[:, :wo]


@jax.jit
def _conv_block(x_nchw, w1_oihw, g1, b1, w2_oihw, g2, b2):
    N, Cin, H, W = x_nchw.shape
    Cmid = w1_oihw.shape[0]
    Cout = w2_oihw.shape[0]
    KW = w1_oihw.shape[3]
    Wo1 = W - (KW - 1)
    Wo2 = Wo1 - (KW - 1)
    P = N * H * W
    count1 = float(N * H * Wo1)
    count2 = float(N * H * Wo2)

    HB1 = 8
    nblk1 = H // HB1
    L1 = HB1 * W * N
    HB3 = 8
    nblk3 = H // HB3
    L3 = HB3 * W * N

    x_t = jnp.transpose(x_nchw, (1, 2, 3, 0))
    w1_cat = jnp.transpose(w1_oihw[:, :, 0, :], (0, 2, 1)).reshape(Cmid, KW * Cin)
    w2_cat = jnp.transpose(w2_oihw[:, :, 0, :], (0, 2, 1)).reshape(Cout, KW * Cmid)
    gb1 = jnp.stack([g1, b1], axis=1)
    gb2 = jnp.stack([g2, b2], axis=1)

    cparams = pltpu.CompilerParams(
        dimension_semantics=("parallel",),
        vmem_limit_bytes=_VMEM_LIMIT,
    )

    def stat_out_spec(c):
        return pl.BlockSpec((1, c, 1), lambda i: (i, 0, 0))

    def stat_in_spec(nblk, c):
        return pl.BlockSpec((nblk, c, 1), lambda i: (0, 0, 0))

    def full2d_spec(r, c):
        return pl.BlockSpec((r, c), lambda i: (0, 0))

    h1, s1, q1 = pl.pallas_call(
        functools.partial(_p1_kernel, kw=KW, w_img=W, wo=Wo1, n=N),
        grid=(nblk1,),
        in_specs=[
            pl.BlockSpec((Cin, HB1, W, N), lambda i: (0, i, 0, 0)),
            full2d_spec(Cmid, KW * Cin),
        ],
        out_specs=(
            pl.BlockSpec((Cmid, L1), lambda i: (0, i)),
            stat_out_spec(Cmid),
            stat_out_spec(Cmid),
        ),
        out_shape=(
            jax.ShapeDtypeStruct((Cmid, P), jnp.bfloat16),
            jax.ShapeDtypeStruct((nblk1, Cmid, 1), jnp.float32),
            jax.ShapeDtypeStruct((nblk1, Cmid, 1), jnp.float32),
        ),
        compiler_params=cparams,
    )(x_t, w1_cat)

    s2, q2 = pl.pallas_call(
        functools.partial(_p2_kernel, kw=KW, w_img=W, wo=Wo2, n=N,
                          count1=count1),
        grid=(nblk1,),
        in_specs=[
            pl.BlockSpec((Cmid, L1), lambda i: (0, i)),
            stat_in_spec(nblk1, Cmid),
            stat_in_spec(nblk1, Cmid),
            full2d_spec(Cmid, 2),
            full2d_spec(Cout, KW * Cmid),
        ],
        out_specs=(stat_out_spec(Cout), stat_out_spec(Cout)),
        out_shape=(
            jax.ShapeDtypeStruct((nblk1, Cout, 1), jnp.float32),
            jax.ShapeDtypeStruct((nblk1, Cout, 1), jnp.float32),
        ),
        compiler_params=cparams,
    )(h1, s1, q1, gb1, w2_cat)

    out_t = pl.pallas_call(
        functools.partial(_p3_kernel, kw=KW, w_img=W, wo=Wo2, n=N, hb=HB3,
                          count1=count1, count2=count2),
        grid=(nblk3,),
        in_specs=[
            pl.BlockSpec((Cmid, L3), lambda i: (0, i)),
            stat_in_spec(nblk1, Cmid),
            stat_in_spec(nblk1, Cmid),
            full2d_spec(Cmid, 2),
            full2d_spec(Cout, KW * Cmid),
            stat_in_spec(nblk1, Cout),
            stat_in_spec(nblk1, Cout),
            full2d_spec(Cout, 2),
        ],
        out_specs=pl.BlockSpec((Cout, Wo2, HB3, N), lambda i: (0, 0, i, 0)),
        out_shape=jax.ShapeDtypeStruct((Cout, Wo2, H, N), jnp.float32),
        compiler_params=cparams,
    )(h1, s1, q1, gb1, w2_cat, s2, q2, gb2)

    return jnp.transpose(out_t, (3, 0, 2, 1))


def kernel(x_nchw, w1_oihw, g1, b1, w2_oihw, g2, b2):
    return _conv_block(x_nchw, w1_oihw, g1, b1, w2_oihw, g2, b2)
